# baseline (speedup 1.0000x reference)
import jax
import jax.numpy as jnp
from jax.experimental import pallas as pl

B = 2
WAY = 5
SHOT = 5
Q_NUM = 75
DIM = 512
H = 10
W = 10
HW = H * W
NEIGHBOR_K = 5
EPS = 1e-5

QROWS = Q_NUM * HW
SROWS = SHOT * HW
SPAD = 512
NCH = 6
CH = QROWS // NCH
NEG = float("-inf")


def _prep_body(sT_ref, sTn_ref, pT_ref):
    sT = sT_ref[0]
    sn2 = jnp.sum(sT * sT, axis=0, keepdims=True)
    sinv = jnp.where(sn2 > 0.0, 1.0 / jnp.sqrt(sn2), 0.0)
    sTn = sT * sinv
    sTn_ref[0] = sTn
    jj = jax.lax.broadcasted_iota(jnp.int32, (WAY * SPAD, WAY), 0)
    wc = jax.lax.broadcasted_iota(jnp.int32, (WAY * SPAD, WAY), 1)
    pool_s = jnp.where(jj // SPAD == wc, 1.0 / SROWS, 0.0)
    pT_ref[0] = jnp.dot(sTn, pool_s, preferred_element_type=jnp.float32, precision=jax.lax.Precision.HIGHEST)


def _main_body(q_ref, sTn_ref, pT_ref, l_ref, s_ref):
    c = pl.program_id(1)
    qc = q_ref[0, 0]
    sTn = sTn_ref[0]

    qn2 = jnp.sum(qc * qc, axis=1, keepdims=True)
    qscale = 1.0 / jnp.sqrt(qn2)
    qp = jnp.dot(qc, pT_ref[0], preferred_element_type=jnp.float32, precision=jax.lax.Precision.HIGHEST) * qscale

    lane = jax.lax.broadcasted_iota(jnp.int32, (CH, SPAD), 1)
    valid = lane < SROWS
    s5all = jnp.zeros((CH, WAY), dtype=jnp.float32)
    for w in range(WAY):
        x = jnp.dot(qc, sTn[:, w * SPAD:(w + 1) * SPAD],
                    preferred_element_type=jnp.float32, precision=jax.lax.Precision.HIGHEST)
        x = jnp.where(valid, x, NEG)
        s5 = jnp.zeros((CH, 1), dtype=jnp.float32)
        rem = jnp.full((CH, 1), float(NEIGHBOR_K), dtype=jnp.float32)
        for _ in range(NEIGHBOR_K):
            m = jnp.max(x, axis=1, keepdims=True)
            eq = x == m
            cnt = jnp.sum(eq.astype(jnp.float32), axis=1, keepdims=True)
            take = jnp.minimum(cnt, rem)
            s5 = s5 + m * take
            rem = rem - take
            x = jnp.where(eq, NEG, x)
        onehot = (jax.lax.broadcasted_iota(jnp.int32, (1, WAY), 1) == w)
        s5all = s5all + (s5 * qscale) * onehot.astype(jnp.float32)

    rr = jax.lax.broadcasted_iota(jnp.int32, (Q_NUM, CH), 0)
    cc = jax.lax.broadcasted_iota(jnp.int32, (Q_NUM, CH), 1)
    pool = jnp.where((cc + c * CH) // HW == rr, 1.0, 0.0)

    @pl.when(c == 0)
    def _():
        l_ref[0] = jnp.zeros((Q_NUM, WAY), dtype=jnp.float32)
        s_ref[0] = jnp.zeros((Q_NUM, WAY), dtype=jnp.float32)

    l_ref[0] += jnp.dot(pool, qp, preferred_element_type=jnp.float32, precision=jax.lax.Precision.HIGHEST)
    s_ref[0] += jnp.dot(pool, s5all, preferred_element_type=jnp.float32, precision=jax.lax.Precision.HIGHEST)


def _post_body(l_ref, s_ref, gc_ref, gs_ref, bc_ref, bs_ref, fw_ref, out_ref):
    logits = l_ref[0] / HW
    sim = s_ref[0] / (HW * NEIGHBOR_K)
    mu_c = jnp.mean(logits, axis=0, keepdims=True)
    var_c = jnp.mean((logits - mu_c) ** 2, axis=0, keepdims=True)
    norm_c = (logits - mu_c) / jnp.sqrt(var_c + EPS) * gc_ref[...] + bc_ref[...]
    mu_s = jnp.mean(sim, axis=0, keepdims=True)
    var_s = jnp.mean((sim - mu_s) ** 2, axis=0, keepdims=True)
    norm_s = (sim - mu_s) / jnp.sqrt(var_s + EPS) * gs_ref[...] + bs_ref[...]
    out_ref[0] = fw_ref[:, 0:1] * norm_c + fw_ref[:, 1:2] * norm_s


def kernel(x_shot, x_query, bn_gamma, bn_beta, fc_weight):
    q4 = jnp.transpose(x_query, (0, 1, 3, 4, 2)).reshape(B, NCH, CH, DIM)
    s4 = jnp.transpose(x_shot, (0, 1, 2, 4, 5, 3)).reshape(B, WAY, SROWS, DIM)
    s4 = jnp.pad(s4, ((0, 0), (0, 0), (0, SPAD - SROWS), (0, 0)))
    sT = jnp.transpose(s4, (0, 3, 1, 2)).reshape(B, DIM, WAY * SPAD)

    gc = bn_gamma[:WAY].reshape(1, WAY)
    gs = bn_gamma[WAY:].reshape(1, WAY)
    bc = bn_beta[:WAY].reshape(1, WAY)
    bs = bn_beta[WAY:].reshape(1, WAY)
    fw = fc_weight.reshape(1, 2)

    sTn, pT = pl.pallas_call(
        _prep_body,
        grid=(B,),
        in_specs=[pl.BlockSpec((1, DIM, WAY * SPAD), lambda b: (b, 0, 0))],
        out_specs=[
            pl.BlockSpec((1, DIM, WAY * SPAD), lambda b: (b, 0, 0)),
            pl.BlockSpec((1, DIM, WAY), lambda b: (b, 0, 0)),
        ],
        out_shape=[
            jax.ShapeDtypeStruct((B, DIM, WAY * SPAD), jnp.float32),
            jax.ShapeDtypeStruct((B, DIM, WAY), jnp.float32),
        ],
    )(sT)

    lsum, ssum = pl.pallas_call(
        _main_body,
        grid=(B, NCH),
        in_specs=[
            pl.BlockSpec((1, 1, CH, DIM), lambda b, c: (b, c, 0, 0)),
            pl.BlockSpec((1, DIM, WAY * SPAD), lambda b, c: (b, 0, 0)),
            pl.BlockSpec((1, DIM, WAY), lambda b, c: (b, 0, 0)),
        ],
        out_specs=[
            pl.BlockSpec((1, Q_NUM, WAY), lambda b, c: (b, 0, 0)),
            pl.BlockSpec((1, Q_NUM, WAY), lambda b, c: (b, 0, 0)),
        ],
        out_shape=[
            jax.ShapeDtypeStruct((B, Q_NUM, WAY), jnp.float32),
            jax.ShapeDtypeStruct((B, Q_NUM, WAY), jnp.float32),
        ],
    )(q4, sTn, pT)

    small = lambda shp: pl.BlockSpec(shp, lambda b: (0,) * len(shp))
    out = pl.pallas_call(
        _post_body,
        grid=(B,),
        in_specs=[
            pl.BlockSpec((1, Q_NUM, WAY), lambda b: (b, 0, 0)),
            pl.BlockSpec((1, Q_NUM, WAY), lambda b: (b, 0, 0)),
            small((1, WAY)), small((1, WAY)), small((1, WAY)), small((1, WAY)),
            small((1, 2)),
        ],
        out_specs=pl.BlockSpec((1, Q_NUM, WAY), lambda b: (b, 0, 0)),
        out_shape=jax.ShapeDtypeStruct((B, Q_NUM, WAY), jnp.float32),
    )(lsum, ssum, gc, gs, bc, bs, fw)
    return out


# speedup vs baseline: 1.0732x; 1.0732x over previous
import jax
import jax.numpy as jnp
from jax.experimental import pallas as pl

B = 2
WAY = 5
SHOT = 5
Q_NUM = 75
DIM = 512
H = 10
W = 10
HW = H * W
NEIGHBOR_K = 5
EPS = 1e-5

QROWS = Q_NUM * HW
SROWS = SHOT * HW
SPAD = 512
NCH = 15
CH = QROWS // NCH
NEG = float("-inf")


def _prep_body(sT_ref, sTn_ref, pT_ref):
    sT = sT_ref[0]
    sn2 = jnp.sum(sT * sT, axis=0, keepdims=True)
    sinv = jnp.where(sn2 > 0.0, 1.0 / jnp.sqrt(sn2), 0.0)
    sTn = sT * sinv
    sTn_ref[0] = sTn
    jj = jax.lax.broadcasted_iota(jnp.int32, (WAY * SPAD, WAY), 0)
    wc = jax.lax.broadcasted_iota(jnp.int32, (WAY * SPAD, WAY), 1)
    pool_s = jnp.where(jj // SPAD == wc, 1.0 / SROWS, 0.0)
    pT_ref[0] = jnp.dot(sTn, pool_s, preferred_element_type=jnp.float32, precision=jax.lax.Precision.HIGHEST)


def _main_body(q_ref, sTn_ref, pT_ref, l_ref, s_ref):
    c = pl.program_id(1)
    qc = q_ref[0, 0]
    sTn = sTn_ref[0]

    qn2 = jnp.sum(qc * qc, axis=1, keepdims=True)
    qscale = 1.0 / jnp.sqrt(qn2)
    qp = jnp.dot(qc, pT_ref[0], preferred_element_type=jnp.float32, precision=jax.lax.Precision.HIGHEST) * qscale

    lane = jax.lax.broadcasted_iota(jnp.int32, (CH, SPAD), 1)
    valid = lane < SROWS
    s5all = jnp.zeros((CH, WAY), dtype=jnp.float32)
    for w in range(WAY):
        x = jnp.dot(qc, sTn[:, w * SPAD:(w + 1) * SPAD],
                    preferred_element_type=jnp.float32, precision=jax.lax.Precision.HIGHEST)
        x = jnp.where(valid, x, NEG)
        s5 = jnp.zeros((CH, 1), dtype=jnp.float32)
        rem = jnp.full((CH, 1), float(NEIGHBOR_K), dtype=jnp.float32)
        for _ in range(NEIGHBOR_K):
            m = jnp.max(x, axis=1, keepdims=True)
            eq = x == m
            cnt = jnp.sum(eq.astype(jnp.float32), axis=1, keepdims=True)
            take = jnp.minimum(cnt, rem)
            s5 = s5 + m * take
            rem = rem - take
            x = jnp.where(eq, NEG, x)
        onehot = (jax.lax.broadcasted_iota(jnp.int32, (1, WAY), 1) == w)
        s5all = s5all + (s5 * qscale) * onehot.astype(jnp.float32)

    rr = jax.lax.broadcasted_iota(jnp.int32, (Q_NUM, CH), 0)
    cc = jax.lax.broadcasted_iota(jnp.int32, (Q_NUM, CH), 1)
    pool = jnp.where((cc + c * CH) // HW == rr, 1.0, 0.0)

    @pl.when(c == 0)
    def _():
        l_ref[0] = jnp.zeros((Q_NUM, WAY), dtype=jnp.float32)
        s_ref[0] = jnp.zeros((Q_NUM, WAY), dtype=jnp.float32)

    l_ref[0] += jnp.dot(pool, qp, preferred_element_type=jnp.float32, precision=jax.lax.Precision.HIGHEST)
    s_ref[0] += jnp.dot(pool, s5all, preferred_element_type=jnp.float32, precision=jax.lax.Precision.HIGHEST)


def _post_body(l_ref, s_ref, gc_ref, gs_ref, bc_ref, bs_ref, fw_ref, out_ref):
    logits = l_ref[0] / HW
    sim = s_ref[0] / (HW * NEIGHBOR_K)
    mu_c = jnp.mean(logits, axis=0, keepdims=True)
    var_c = jnp.mean((logits - mu_c) ** 2, axis=0, keepdims=True)
    norm_c = (logits - mu_c) / jnp.sqrt(var_c + EPS) * gc_ref[...] + bc_ref[...]
    mu_s = jnp.mean(sim, axis=0, keepdims=True)
    var_s = jnp.mean((sim - mu_s) ** 2, axis=0, keepdims=True)
    norm_s = (sim - mu_s) / jnp.sqrt(var_s + EPS) * gs_ref[...] + bs_ref[...]
    out_ref[0] = fw_ref[:, 0:1] * norm_c + fw_ref[:, 1:2] * norm_s


def kernel(x_shot, x_query, bn_gamma, bn_beta, fc_weight):
    q4 = jnp.transpose(x_query, (0, 1, 3, 4, 2)).reshape(B, NCH, CH, DIM)
    s4 = jnp.transpose(x_shot, (0, 1, 2, 4, 5, 3)).reshape(B, WAY, SROWS, DIM)
    s4 = jnp.pad(s4, ((0, 0), (0, 0), (0, SPAD - SROWS), (0, 0)))
    sT = jnp.transpose(s4, (0, 3, 1, 2)).reshape(B, DIM, WAY * SPAD)

    gc = bn_gamma[:WAY].reshape(1, WAY)
    gs = bn_gamma[WAY:].reshape(1, WAY)
    bc = bn_beta[:WAY].reshape(1, WAY)
    bs = bn_beta[WAY:].reshape(1, WAY)
    fw = fc_weight.reshape(1, 2)

    sTn, pT = pl.pallas_call(
        _prep_body,
        grid=(B,),
        in_specs=[pl.BlockSpec((1, DIM, WAY * SPAD), lambda b: (b, 0, 0))],
        out_specs=[
            pl.BlockSpec((1, DIM, WAY * SPAD), lambda b: (b, 0, 0)),
            pl.BlockSpec((1, DIM, WAY), lambda b: (b, 0, 0)),
        ],
        out_shape=[
            jax.ShapeDtypeStruct((B, DIM, WAY * SPAD), jnp.float32),
            jax.ShapeDtypeStruct((B, DIM, WAY), jnp.float32),
        ],
    )(sT)

    lsum, ssum = pl.pallas_call(
        _main_body,
        grid=(B, NCH),
        in_specs=[
            pl.BlockSpec((1, 1, CH, DIM), lambda b, c: (b, c, 0, 0)),
            pl.BlockSpec((1, DIM, WAY * SPAD), lambda b, c: (b, 0, 0)),
            pl.BlockSpec((1, DIM, WAY), lambda b, c: (b, 0, 0)),
        ],
        out_specs=[
            pl.BlockSpec((1, Q_NUM, WAY), lambda b, c: (b, 0, 0)),
            pl.BlockSpec((1, Q_NUM, WAY), lambda b, c: (b, 0, 0)),
        ],
        out_shape=[
            jax.ShapeDtypeStruct((B, Q_NUM, WAY), jnp.float32),
            jax.ShapeDtypeStruct((B, Q_NUM, WAY), jnp.float32),
        ],
    )(q4, sTn, pT)

    small = lambda shp: pl.BlockSpec(shp, lambda b: (0,) * len(shp))
    out = pl.pallas_call(
        _post_body,
        grid=(B,),
        in_specs=[
            pl.BlockSpec((1, Q_NUM, WAY), lambda b: (b, 0, 0)),
            pl.BlockSpec((1, Q_NUM, WAY), lambda b: (b, 0, 0)),
            small((1, WAY)), small((1, WAY)), small((1, WAY)), small((1, WAY)),
            small((1, 2)),
        ],
        out_specs=pl.BlockSpec((1, Q_NUM, WAY), lambda b: (b, 0, 0)),
        out_shape=jax.ShapeDtypeStruct((B, Q_NUM, WAY), jnp.float32),
    )(lsum, ssum, gc, gs, bc, bs, fw)
    return out


# speedup vs baseline: 1.2174x; 1.1344x over previous
import jax
import jax.numpy as jnp
from jax.experimental import pallas as pl

B = 2
WAY = 5
SHOT = 5
Q_NUM = 75
DIM = 512
H = 10
W = 10
HW = H * W
NEIGHBOR_K = 5
EPS = 1e-5

QROWS = Q_NUM * HW
SROWS = SHOT * HW
SPAD = 512
NCH = 10
CH = QROWS // NCH
NEG = float("-inf")
HP = jax.lax.Precision.HIGHEST


def _prep_body(sT_ref, sTn_ref):
    sT = sT_ref[0]
    sn2 = jnp.sum(sT * sT, axis=0, keepdims=True)
    sinv = jnp.where(sn2 > 0.0, 1.0 / jnp.sqrt(sn2), 0.0)
    sTn = sT * sinv
    jj = jax.lax.broadcasted_iota(jnp.int32, (WAY * SPAD, WAY), 0)
    wc = jax.lax.broadcasted_iota(jnp.int32, (WAY * SPAD, WAY), 1)
    pool_s = jnp.where(jj // SPAD == wc, 1.0 / SROWS, 0.0)
    protosT = jnp.dot(sTn, pool_s, preferred_element_type=jnp.float32,
                      precision=HP)
    sTn_ref[0] = sTn + jnp.pad(
        protosT, ((0, 0), (SROWS, WAY * SPAD - SROWS - WAY)))


def _main_body(q_ref, sTn_ref, p_ref):
    c = pl.program_id(1)
    qc = q_ref[0, 0]
    sTn = sTn_ref[0]

    qn2 = jnp.sum(qc * qc, axis=1, keepdims=True)
    qscale = 1.0 / jnp.sqrt(qn2)

    lane = jax.lax.broadcasted_iota(jnp.int32, (CH, SPAD), 1)
    valid = lane < SROWS
    acc = jnp.zeros((CH, 2 * WAY), dtype=jnp.float32)
    col10 = jax.lax.broadcasted_iota(jnp.int32, (1, 2 * WAY), 1)
    for w in range(WAY):
        x = jnp.dot(qc, sTn[:, w * SPAD:(w + 1) * SPAD],
                    preferred_element_type=jnp.float32, precision=HP)
        if w == 0:
            qp = x[:, SROWS:SROWS + WAY] * qscale
            acc = acc + jnp.pad(qp, ((0, 0), (0, WAY)))
        x = jnp.where(valid, x, NEG)
        s5 = jnp.zeros((CH, 1), dtype=jnp.float32)
        rem = jnp.full((CH, 1), float(NEIGHBOR_K), dtype=jnp.float32)
        for _ in range(NEIGHBOR_K):
            m = jnp.max(x, axis=1, keepdims=True)
            eq = x == m
            cnt = jnp.sum(eq.astype(jnp.float32), axis=1, keepdims=True)
            take = jnp.minimum(cnt, rem)
            s5 = s5 + m * take
            rem = rem - take
            x = jnp.where(eq, NEG, x)
        acc = acc + (s5 * qscale) * (col10 == WAY + w).astype(jnp.float32)

    rr = jax.lax.broadcasted_iota(jnp.int32, (Q_NUM, CH), 0)
    cc = jax.lax.broadcasted_iota(jnp.int32, (Q_NUM, CH), 1)
    pool = jnp.where((cc + c * CH) // HW == rr, 1.0, 0.0)

    @pl.when(c == 0)
    def _():
        p_ref[0] = jnp.zeros((Q_NUM, 2 * WAY), dtype=jnp.float32)

    p_ref[0] += jnp.dot(pool, acc, preferred_element_type=jnp.float32,
                        precision=HP)


def _post_body(p_ref, gc_ref, gs_ref, bc_ref, bs_ref, fw_ref, out_ref):
    ps = p_ref[0]
    logits = ps[:, :WAY] / HW
    sim = ps[:, WAY:] / (HW * NEIGHBOR_K)
    mu_c = jnp.mean(logits, axis=0, keepdims=True)
    var_c = jnp.mean((logits - mu_c) ** 2, axis=0, keepdims=True)
    norm_c = (logits - mu_c) / jnp.sqrt(var_c + EPS) * gc_ref[...] + bc_ref[...]
    mu_s = jnp.mean(sim, axis=0, keepdims=True)
    var_s = jnp.mean((sim - mu_s) ** 2, axis=0, keepdims=True)
    norm_s = (sim - mu_s) / jnp.sqrt(var_s + EPS) * gs_ref[...] + bs_ref[...]
    out_ref[0] = fw_ref[:, 0:1] * norm_c + fw_ref[:, 1:2] * norm_s


def kernel(x_shot, x_query, bn_gamma, bn_beta, fc_weight):
    q4 = jnp.transpose(x_query, (0, 1, 3, 4, 2)).reshape(B, NCH, CH, DIM)
    s4 = jnp.transpose(x_shot, (0, 1, 2, 4, 5, 3)).reshape(B, WAY, SROWS, DIM)
    s4 = jnp.pad(s4, ((0, 0), (0, 0), (0, SPAD - SROWS), (0, 0)))
    sT = jnp.transpose(s4, (0, 3, 1, 2)).reshape(B, DIM, WAY * SPAD)

    gc = bn_gamma[:WAY].reshape(1, WAY)
    gs = bn_gamma[WAY:].reshape(1, WAY)
    bc = bn_beta[:WAY].reshape(1, WAY)
    bs = bn_beta[WAY:].reshape(1, WAY)
    fw = fc_weight.reshape(1, 2)

    sTn, = pl.pallas_call(
        _prep_body,
        grid=(B,),
        in_specs=[pl.BlockSpec((1, DIM, WAY * SPAD), lambda b: (b, 0, 0))],
        out_specs=[pl.BlockSpec((1, DIM, WAY * SPAD), lambda b: (b, 0, 0))],
        out_shape=[jax.ShapeDtypeStruct((B, DIM, WAY * SPAD), jnp.float32)],
    )(sT)

    psum, = pl.pallas_call(
        _main_body,
        grid=(B, NCH),
        in_specs=[
            pl.BlockSpec((1, 1, CH, DIM), lambda b, c: (b, c, 0, 0)),
            pl.BlockSpec((1, DIM, WAY * SPAD), lambda b, c: (b, 0, 0)),
        ],
        out_specs=[pl.BlockSpec((1, Q_NUM, 2 * WAY), lambda b, c: (b, 0, 0))],
        out_shape=[jax.ShapeDtypeStruct((B, Q_NUM, 2 * WAY), jnp.float32)],
    )(q4, sTn)

    small = lambda shp: pl.BlockSpec(shp, lambda b: (0,) * len(shp))
    out = pl.pallas_call(
        _post_body,
        grid=(B,),
        in_specs=[
            pl.BlockSpec((1, Q_NUM, 2 * WAY), lambda b: (b, 0, 0)),
            small((1, WAY)), small((1, WAY)), small((1, WAY)), small((1, WAY)),
            small((1, 2)),
        ],
        out_specs=pl.BlockSpec((1, Q_NUM, WAY), lambda b: (b, 0, 0)),
        out_shape=jax.ShapeDtypeStruct((B, Q_NUM, WAY), jnp.float32),
    )(psum, gc, gs, bc, bs, fw)
    return out
